# baseline (speedup 1.0000x reference)
import functools

import jax
import jax.numpy as jnp
from jax import lax
from jax.experimental import pallas as pl
from jax.experimental.pallas import tpu as pltpu
from jax.experimental.pallas import tpu_sc as plsc

SEQ = 200
BATCH = 4096
EMB = 64
LANES = 16
NUM_CORES = 2
NUM_SUBCORES = 16
NUM_WORKERS = NUM_CORES * NUM_SUBCORES
NB = BATCH // NUM_WORKERS
HALF = SEQ // 2
NBUF = 2
ECHUNKS = EMB // LANES


def _body(table_hbm, xt_hbm, out_hbm, idx_v, rows_v, out_v, sem0, sem1):
    wid = lax.axis_index("s") * NUM_CORES + lax.axis_index("c")
    base = wid * NB
    sems = [sem0, sem1]

    pltpu.sync_copy(xt_hbm.at[pl.ds(base, NB)], idx_v)

    def fire(e, b):
        for h in range(2):
            pltpu.async_copy(
                table_hbm.at[idx_v.at[e, h]], rows_v.at[b, h], sems[b])

    def drain(b):
        for h in range(2):
            pltpu.make_async_copy(
                table_hbm.at[idx_v.at[0, 0]], rows_v.at[b, h], sems[b]).wait()

    for b in range(NBUF):
        fire(b, b)

    inv = jnp.full((LANES,), 1.0 / SEQ, dtype=jnp.float32)

    def outer(g, carry):
        for b in range(NBUF):
            e = g * NBUF + b
            drain(b)
            acc = [jnp.zeros((LANES,), jnp.float32) for _ in range(ECHUNKS)]

            def add_half(h, acc):
                def body(s, acc):
                    return tuple(
                        acc[j] + rows_v[b, h, s, pl.ds(j * LANES, LANES)]
                        for j in range(ECHUNKS))
                return lax.fori_loop(0, HALF, body, tuple(acc))

            acc = add_half(0, acc)
            acc = add_half(1, acc)

            @pl.when(e + NBUF < NB)
            def _():
                fire(e + NBUF, b)

            for j in range(ECHUNKS):
                out_v[e, pl.ds(j * LANES, LANES)] = acc[j] * inv
        return carry

    lax.fori_loop(0, NB // NBUF, outer, 0)

    pltpu.sync_copy(out_v, out_hbm.at[pl.ds(base, NB)])


@jax.jit
def _fast_text(table, xt):
    mesh = plsc.VectorSubcoreMesh(
        core_axis_name="c", subcore_axis_name="s",
        num_cores=NUM_CORES, num_subcores=NUM_SUBCORES)
    grid_kernel = pl.kernel(
        _body,
        out_type=jax.ShapeDtypeStruct((BATCH, EMB), jnp.float32),
        mesh=mesh,
        compiler_params=pltpu.CompilerParams(use_tc_tiling_on_sc=False),
        scratch_types=[
            pltpu.VMEM((NB, 2, HALF), jnp.int32),
            pltpu.VMEM((NBUF, 2, HALF, EMB), jnp.float32),
            pltpu.VMEM((NB, EMB), jnp.float32),
            pltpu.SemaphoreType.DMA,
            pltpu.SemaphoreType.DMA,
        ],
    )
    return grid_kernel(table, xt)


def kernel(x, table):
    xt = jnp.transpose(x).reshape(BATCH, 2, HALF).astype(jnp.int32)
    return _fast_text(table, xt)


# speedup vs baseline: 1.0084x; 1.0084x over previous
import jax
import jax.numpy as jnp
from jax import lax
from jax.experimental import pallas as pl
from jax.experimental.pallas import tpu as pltpu
from jax.experimental.pallas import tpu_sc as plsc

SEQ = 200
BATCH = 4096
EMB = 64
LANES = 16
NUM_CORES = 2
NUM_SUBCORES = 16
NUM_WORKERS = NUM_CORES * NUM_SUBCORES
NB = BATCH // NUM_WORKERS
NBUF = 4
ECHUNKS = EMB // LANES
IU = 4


def _body(table_hbm, x_hbm, out_hbm, idx_v, rows_v, acc_v,
          sem0, sem1, sem2, sem3):
    wid = lax.axis_index("s") * NUM_CORES + lax.axis_index("c")
    base = wid * NB
    sems = [sem0, sem1, sem2, sem3]

    pltpu.sync_copy(x_hbm.at[pl.ds(0, SEQ), pl.ds(base, NB)], idx_v)

    zero = jnp.zeros((LANES,), jnp.float32)

    def zero_body(i, carry):
        for j in range(ECHUNKS):
            acc_v[i, pl.ds(j * LANES, LANES)] = zero
        return carry

    lax.fori_loop(0, NB, zero_body, 0)

    def fire(s, b):
        pltpu.async_copy(table_hbm.at[idx_v.at[s]], rows_v.at[b], sems[b])

    def drain(b):
        pltpu.make_async_copy(
            table_hbm.at[idx_v.at[0]], rows_v.at[b], sems[b]).wait()

    for b in range(NBUF):
        fire(b, b)

    def outer(g, carry):
        for b in range(NBUF):
            s = g * NBUF + b
            drain(b)

            def add_body(t, c):
                for u in range(IU):
                    i = t * IU + u
                    for j in range(ECHUNKS):
                        plsc.addupdate(
                            acc_v.at[i, pl.ds(j * LANES, LANES)],
                            rows_v[b, i, pl.ds(j * LANES, LANES)])
                return c

            lax.fori_loop(0, NB // IU, add_body, 0)

            @pl.when(s + NBUF < SEQ)
            def _():
                fire(s + NBUF, b)
        return carry

    lax.fori_loop(0, SEQ // NBUF, outer, 0)

    inv = jnp.full((LANES,), 1.0 / SEQ, dtype=jnp.float32)

    def scale_body(i, carry):
        for j in range(ECHUNKS):
            sl = pl.ds(j * LANES, LANES)
            acc_v[i, sl] = acc_v[i, sl] * inv
        return carry

    lax.fori_loop(0, NB, scale_body, 0)
    pltpu.sync_copy(acc_v, out_hbm.at[pl.ds(base, NB)])


@jax.jit
def _fast_text(table, x):
    mesh = plsc.VectorSubcoreMesh(
        core_axis_name="c", subcore_axis_name="s",
        num_cores=NUM_CORES, num_subcores=NUM_SUBCORES)
    grid_kernel = pl.kernel(
        _body,
        out_type=jax.ShapeDtypeStruct((BATCH, EMB), jnp.float32),
        mesh=mesh,
        compiler_params=pltpu.CompilerParams(use_tc_tiling_on_sc=False),
        scratch_types=[
            pltpu.VMEM((SEQ, NB), jnp.int32),
            pltpu.VMEM((NBUF, NB, EMB), jnp.float32),
            pltpu.VMEM((NB, EMB), jnp.float32),
            pltpu.SemaphoreType.DMA,
            pltpu.SemaphoreType.DMA,
            pltpu.SemaphoreType.DMA,
            pltpu.SemaphoreType.DMA,
        ],
    )
    return grid_kernel(table, x)


def kernel(x, table):
    return _fast_text(table, x.astype(jnp.int32))
